# baseline (speedup 1.0000x reference)
import functools

import jax
import jax.numpy as jnp
import numpy as np
from jax.experimental import pallas as pl
from jax.experimental.pallas import tpu as pltpu

T = 500
STRUC_N = 517
SEQ_N = 33
STRUC_MASK = 516
SEQ_MASK = 32
EPS = 1e-10
_NEG = np.int32(-(2 ** 31))
_BIG = np.int32(2 ** 30)


def _threefry_bits(k1, k2, x1):
    ks0 = k1
    ks1 = k2
    ks2 = k1 ^ k2 ^ jnp.uint32(0x1BD11BDA)
    ks = (ks0, ks1, ks2)
    x0 = jnp.zeros_like(x1) + ks0
    x1 = x1 + ks1
    rot = (13, 15, 26, 6, 17, 29, 16, 24)
    rounds = (rot[0:4], rot[4:8], rot[0:4], rot[4:8], rot[0:4])
    for i, chunk in enumerate(rounds):
        for r in chunk:
            x0 = x0 + x1
            x1 = (x1 << jnp.uint32(r)) | (x1 >> jnp.uint32(32 - r))
            x1 = x0 ^ x1
        x0 = x0 + ks[(i + 1) % 3]
        x1 = x1 + ks[(i + 2) % 3] + jnp.uint32(i + 1)
    return x0 ^ x1


def _sample_body(key_ref, x0_ref, out_ref, *, n_cls, n_pad, mask_idx, lanes):
    p = pl.program_id(0)
    k1 = jax.lax.bitcast_convert_type(key_ref[0], jnp.uint32)
    k2 = jax.lax.bitcast_convert_type(key_ref[1], jnp.uint32)
    c = jax.lax.broadcasted_iota(jnp.int32, (n_pad, lanes), 0)
    lane = jax.lax.broadcasted_iota(jnp.int32, (n_pad, lanes), 1)
    row = p * lanes + lane
    i = (row * n_cls + c).astype(jnp.uint32)
    bits = _threefry_bits(k1, k2, i)
    biased = jax.lax.bitcast_convert_type(bits ^ jnp.uint32(0x80000000),
                                          jnp.int32)
    x0 = x0_ref[0]
    is_x0 = c == x0
    is_mask = c == mask_idx
    excl = is_x0 | is_mask | (c >= n_cls)
    b_eps = jnp.where(excl, _NEG, biased)
    eps_max = jnp.max(b_eps, axis=0, keepdims=True)
    idx_eps = jnp.min(jnp.where(b_eps == eps_max, c, _BIG), axis=0,
                      keepdims=True)
    b_x0 = jnp.max(jnp.where(is_x0, biased, _NEG), axis=0, keepdims=True)
    b_mask = jnp.max(jnp.where(is_mask, biased, _NEG), axis=0, keepdims=True)
    out_ref[0, 0:1, :] = b_x0
    out_ref[0, 1:2, :] = b_mask
    out_ref[0, 2:3, :] = eps_max
    out_ref[0, 3:4, :] = idx_eps
    out_ref[0, 4:8, :] = jnp.zeros((4, lanes), jnp.int32)


def _candidates(x_flat, key_data, n_cls, n_pad, mask_idx, lanes):
    rows = x_flat.shape[0]
    grid = rows // lanes
    x_in = x_flat.reshape(grid, 1, lanes)
    body = functools.partial(_sample_body, n_cls=n_cls, n_pad=n_pad,
                             mask_idx=mask_idx, lanes=lanes)
    out = pl.pallas_call(
        body,
        grid=(grid,),
        in_specs=[
            pl.BlockSpec(memory_space=pltpu.SMEM),
            pl.BlockSpec((1, 1, lanes), lambda p: (p, 0, 0)),
        ],
        out_specs=pl.BlockSpec((1, 8, lanes), lambda p: (p, 0, 0)),
        out_shape=jax.ShapeDtypeStruct((grid, 8, lanes), jnp.int32),
        compiler_params=pltpu.CompilerParams(
            dimension_semantics=("parallel",)),
    )(key_data.astype(jnp.int32), x_in)
    unbias = lambda b: jax.lax.bitcast_convert_type(b, jnp.uint32) ^ jnp.uint32(
        0x80000000)
    b_x0 = unbias(out[:, 0, :].reshape(rows))
    b_mask = unbias(out[:, 1, :].reshape(rows))
    b_eps = unbias(out[:, 2, :].reshape(rows))
    i_eps = out[:, 3, :].reshape(rows)
    return b_x0, b_mask, b_eps, i_eps


def _gumbel_from_bits(bits):
    tiny = jnp.float32(jnp.finfo(jnp.float32).tiny)
    fb = (bits >> jnp.uint32(9)) | jnp.uint32(0x3F800000)
    floats = jax.lax.bitcast_convert_type(fb, jnp.float32) - jnp.float32(1.0)
    u = jnp.maximum(tiny, floats * (jnp.float32(1.0) - tiny) + tiny)
    return -jnp.log(-jnp.log(u))


def _finish(b_x0, b_mask, b_eps, i_eps, x_flat, a_flat, mask_idx):
    eq = x_flat == mask_idx
    one_minus_a = jnp.float32(1.0) - a_flat
    p_x0 = a_flat + jnp.where(eq, one_minus_a, jnp.float32(0.0))
    p_m = jnp.where(eq, a_flat + one_minus_a, one_minus_a)
    v1 = _gumbel_from_bits(b_x0) + jnp.log(p_x0 + EPS)
    v2 = _gumbel_from_bits(b_mask) + jnp.log(p_m + EPS)
    v3 = _gumbel_from_bits(b_eps) + jnp.log(jnp.float32(0.0) + EPS)
    i1 = x_flat
    i2 = jnp.full_like(x_flat, mask_idx)
    best_v, best_i = v1, i1
    upd = (v2 > best_v) | ((v2 == best_v) & (i2 < best_i))
    best_v = jnp.where(upd, v2, best_v)
    best_i = jnp.where(upd, i2, best_i)
    upd = (v3 > best_v) | ((v3 == best_v) & (i_eps < best_i))
    best_i = jnp.where(upd, i_eps, best_i)
    return best_i


def kernel(structure, sequence, t):
    t_idx = jnp.arange(T + 1, dtype=jnp.float32)
    beta = 1.0 / (T - t_idx + 1.0)
    alpha = jnp.cumprod(1.0 - beta)
    key = jax.random.key(42)
    ks, kq = jax.random.split(key)
    kd_s = jax.random.key_data(ks)
    kd_q = jax.random.key_data(kq)
    B, L = structure.shape
    a_flat = jnp.repeat(alpha[t], L)
    outs = []
    for x, kd, n_cls, n_pad, mask_idx, lanes in (
            (structure, kd_s, STRUC_N, 520, STRUC_MASK, 512),
            (sequence, kd_q, SEQ_N, 40, SEQ_MASK, 2048)):
        x_flat = x.reshape(-1).astype(jnp.int32)
        cands = _candidates(x_flat, kd, n_cls, n_pad, mask_idx, lanes)
        tok = _finish(*cands, x_flat, a_flat, mask_idx)
        outs.append(tok.reshape(B, L))
    return outs[0], outs[1], t


# speedup vs baseline: 1.0106x; 1.0106x over previous
import functools

import jax
import jax.numpy as jnp
import numpy as np
from jax.experimental import pallas as pl
from jax.experimental.pallas import tpu as pltpu

T = 500
STRUC_N = 517
SEQ_N = 33
STRUC_MASK = 516
SEQ_MASK = 32
EPS = 1e-10
_NEG = np.int32(-(2 ** 31))
_BIG = np.int32(2 ** 30)


def _threefry_bits(k1, k2, x1):
    ks0 = k1
    ks1 = k2
    ks2 = k1 ^ k2 ^ jnp.uint32(0x1BD11BDA)
    ks = (ks0, ks1, ks2)
    x0 = jnp.zeros_like(x1) + ks0
    x1 = x1 + ks1
    rot = (13, 15, 26, 6, 17, 29, 16, 24)
    rounds = (rot[0:4], rot[4:8], rot[0:4], rot[4:8], rot[0:4])
    for i, chunk in enumerate(rounds):
        for r in chunk:
            x0 = x0 + x1
            x1 = (x1 << jnp.uint32(r)) | (x1 >> jnp.uint32(32 - r))
            x1 = x0 ^ x1
        x0 = x0 + ks[(i + 1) % 3]
        x1 = x1 + ks[(i + 2) % 3] + jnp.uint32(i + 1)
    return x0 ^ x1


def _sample_body(key_ref, x0_ref, out_ref, *, n_cls, n_pad, mask_idx, lanes):
    p = pl.program_id(0)
    k1 = jax.lax.bitcast_convert_type(key_ref[0], jnp.uint32)
    k2 = jax.lax.bitcast_convert_type(key_ref[1], jnp.uint32)
    c = jax.lax.broadcasted_iota(jnp.int32, (n_pad, lanes), 0)
    lane = jax.lax.broadcasted_iota(jnp.int32, (n_pad, lanes), 1)
    row = p * lanes + lane
    i = (row * n_cls + c).astype(jnp.uint32)
    bits = _threefry_bits(k1, k2, i)
    biased = jax.lax.bitcast_convert_type(bits ^ jnp.uint32(0x80000000),
                                          jnp.int32)
    x0 = x0_ref[0]
    is_x0 = c == x0
    is_mask = c == mask_idx
    excl = is_x0 | is_mask | (c >= n_cls)
    b_eps = jnp.where(excl, _NEG, biased)
    eps_max = jnp.max(b_eps, axis=0, keepdims=True)
    idx_eps = jnp.min(jnp.where(b_eps == eps_max, c, _BIG), axis=0,
                      keepdims=True)
    b_x0 = jnp.max(jnp.where(is_x0, biased, _NEG), axis=0, keepdims=True)
    b_mask = jnp.max(jnp.where(is_mask, biased, _NEG), axis=0, keepdims=True)
    out_ref[0, 0:1, :] = b_x0
    out_ref[0, 1:2, :] = b_mask
    out_ref[0, 2:3, :] = eps_max
    out_ref[0, 3:4, :] = idx_eps
    out_ref[0, 4:8, :] = jnp.zeros((4, lanes), jnp.int32)


def _candidates(x_flat, key_data, n_cls, n_pad, mask_idx, lanes):
    rows = x_flat.shape[0]
    grid = rows // lanes
    x_in = x_flat.reshape(grid, 1, lanes)
    body = functools.partial(_sample_body, n_cls=n_cls, n_pad=n_pad,
                             mask_idx=mask_idx, lanes=lanes)
    out = pl.pallas_call(
        body,
        grid=(grid,),
        in_specs=[
            pl.BlockSpec(memory_space=pltpu.SMEM),
            pl.BlockSpec((1, 1, lanes), lambda p: (p, 0, 0)),
        ],
        out_specs=pl.BlockSpec((1, 8, lanes), lambda p: (p, 0, 0)),
        out_shape=jax.ShapeDtypeStruct((grid, 8, lanes), jnp.int32),
        compiler_params=pltpu.CompilerParams(
            dimension_semantics=("parallel",)),
    )(key_data.astype(jnp.int32), x_in)
    unbias = lambda b: jax.lax.bitcast_convert_type(b, jnp.uint32) ^ jnp.uint32(
        0x80000000)
    b_x0 = unbias(out[:, 0, :].reshape(rows))
    b_mask = unbias(out[:, 1, :].reshape(rows))
    b_eps = unbias(out[:, 2, :].reshape(rows))
    i_eps = out[:, 3, :].reshape(rows)
    return b_x0, b_mask, b_eps, i_eps


def _gumbel_from_bits(bits):
    tiny = jnp.float32(jnp.finfo(jnp.float32).tiny)
    fb = (bits >> jnp.uint32(9)) | jnp.uint32(0x3F800000)
    floats = jax.lax.bitcast_convert_type(fb, jnp.float32) - jnp.float32(1.0)
    u = jnp.maximum(tiny, floats * (jnp.float32(1.0) - tiny) + tiny)
    return -jnp.log(-jnp.log(u))


def _finish(b_x0, b_mask, b_eps, i_eps, x_flat, a_flat, mask_idx):
    eq = x_flat == mask_idx
    one_minus_a = jnp.float32(1.0) - a_flat
    p_x0 = a_flat + jnp.where(eq, one_minus_a, jnp.float32(0.0))
    p_m = jnp.where(eq, a_flat + one_minus_a, one_minus_a)
    v1 = _gumbel_from_bits(b_x0) + jnp.log(p_x0 + EPS)
    v2 = _gumbel_from_bits(b_mask) + jnp.log(p_m + EPS)
    v3 = _gumbel_from_bits(b_eps) + jnp.log(jnp.float32(0.0) + EPS)
    i1 = x_flat
    i2 = jnp.full_like(x_flat, mask_idx)
    best_v, best_i = v1, i1
    upd = (v2 > best_v) | ((v2 == best_v) & (i2 < best_i))
    best_v = jnp.where(upd, v2, best_v)
    best_i = jnp.where(upd, i2, best_i)
    upd = (v3 > best_v) | ((v3 == best_v) & (i_eps < best_i))
    best_i = jnp.where(upd, i_eps, best_i)
    return best_i


def kernel(structure, sequence, t):
    t_idx = jnp.arange(T + 1, dtype=jnp.float32)
    beta = 1.0 / (T - t_idx + 1.0)
    alpha = jnp.cumprod(1.0 - beta)
    key = jax.random.key(42)
    ks, kq = jax.random.split(key)
    kd_s = jax.random.key_data(ks)
    kd_q = jax.random.key_data(kq)
    B, L = structure.shape
    a_flat = jnp.repeat(alpha[t], L)
    outs = []
    for x, kd, n_cls, n_pad, mask_idx, lanes in (
            (structure, kd_s, STRUC_N, 520, STRUC_MASK, 256),
            (sequence, kd_q, SEQ_N, 40, SEQ_MASK, 512)):
        x_flat = x.reshape(-1).astype(jnp.int32)
        cands = _candidates(x_flat, kd, n_cls, n_pad, mask_idx, lanes)
        tok = _finish(*cands, x_flat, a_flat, mask_idx)
        outs.append(tok.reshape(B, L))
    return outs[0], outs[1], t


# speedup vs baseline: 1.1209x; 1.1091x over previous
import functools

import jax
import jax.numpy as jnp
import numpy as np
from jax.experimental import pallas as pl
from jax.experimental.pallas import tpu as pltpu

T = 500
STRUC_N = 517
SEQ_N = 33
STRUC_MASK = 516
SEQ_MASK = 32
EPS = 1e-10
_NEG = np.int32(-(2 ** 31))
_BIG = np.int32(2 ** 30)


def _threefry_bits(k1, k2, x1):
    ks0 = k1
    ks1 = k2
    ks2 = k1 ^ k2 ^ jnp.uint32(0x1BD11BDA)
    ks = (ks0, ks1, ks2)
    x0 = jnp.zeros_like(x1) + ks0
    x1 = x1 + ks1
    rot = (13, 15, 26, 6, 17, 29, 16, 24)
    rounds = (rot[0:4], rot[4:8], rot[0:4], rot[4:8], rot[0:4])
    for i, chunk in enumerate(rounds):
        for r in chunk:
            x0 = x0 + x1
            x1 = (x1 << jnp.uint32(r)) | (x1 >> jnp.uint32(32 - r))
            x1 = x0 ^ x1
        x0 = x0 + ks[(i + 1) % 3]
        x1 = x1 + ks[(i + 2) % 3] + jnp.uint32(i + 1)
    return x0 ^ x1


def _sample_body(key_ref, x0_ref, out_ref, *, n_cls, n_pad, mask_idx, lanes):
    p = pl.program_id(0)
    k1 = jax.lax.bitcast_convert_type(key_ref[0], jnp.uint32)
    k2 = jax.lax.bitcast_convert_type(key_ref[1], jnp.uint32)
    c = jax.lax.broadcasted_iota(jnp.int32, (n_pad, lanes), 0)
    lane = jax.lax.broadcasted_iota(jnp.int32, (n_pad, lanes), 1)
    row = p * lanes + lane
    i = (row * n_cls + c).astype(jnp.uint32)
    bits = _threefry_bits(k1, k2, i)
    biased = jax.lax.bitcast_convert_type(bits ^ jnp.uint32(0x80000000),
                                          jnp.int32)
    x0 = x0_ref[0]
    is_x0 = c == x0
    is_mask = c == mask_idx
    excl = is_x0 | is_mask | (c >= n_cls)
    b_eps = jnp.where(excl, _NEG, biased)
    eps_max = jnp.max(b_eps, axis=0, keepdims=True)
    idx_eps = jnp.min(jnp.where(b_eps == eps_max, c, _BIG), axis=0,
                      keepdims=True)
    b_x0 = jnp.max(jnp.where(is_x0, biased, _NEG), axis=0, keepdims=True)
    b_mask = jnp.max(jnp.where(is_mask, biased, _NEG), axis=0, keepdims=True)
    out_ref[0, 0:1, :] = b_x0
    out_ref[0, 1:2, :] = b_mask
    out_ref[0, 2:3, :] = eps_max
    out_ref[0, 3:4, :] = idx_eps
    out_ref[0, 4:8, :] = jnp.zeros((4, lanes), jnp.int32)


def _candidates(x_flat, key_data, n_cls, n_pad, mask_idx, lanes):
    rows = x_flat.shape[0]
    grid = rows // lanes
    x_in = x_flat.reshape(grid, 1, lanes)
    body = functools.partial(_sample_body, n_cls=n_cls, n_pad=n_pad,
                             mask_idx=mask_idx, lanes=lanes)
    out = pl.pallas_call(
        body,
        grid=(grid,),
        in_specs=[
            pl.BlockSpec(memory_space=pltpu.SMEM),
            pl.BlockSpec((1, 1, lanes), lambda p: (p, 0, 0)),
        ],
        out_specs=pl.BlockSpec((1, 8, lanes), lambda p: (p, 0, 0)),
        out_shape=jax.ShapeDtypeStruct((grid, 8, lanes), jnp.int32),
        compiler_params=pltpu.CompilerParams(
            dimension_semantics=("parallel",)),
    )(key_data.astype(jnp.int32), x_in)
    unbias = lambda b: jax.lax.bitcast_convert_type(b, jnp.uint32) ^ jnp.uint32(
        0x80000000)
    b_x0 = unbias(out[:, 0, :].reshape(rows))
    b_mask = unbias(out[:, 1, :].reshape(rows))
    b_eps = unbias(out[:, 2, :].reshape(rows))
    i_eps = out[:, 3, :].reshape(rows)
    return b_x0, b_mask, b_eps, i_eps


def _gumbel_from_bits(bits):
    tiny = jnp.float32(jnp.finfo(jnp.float32).tiny)
    fb = (bits >> jnp.uint32(9)) | jnp.uint32(0x3F800000)
    floats = jax.lax.bitcast_convert_type(fb, jnp.float32) - jnp.float32(1.0)
    u = jnp.maximum(tiny, floats * (jnp.float32(1.0) - tiny) + tiny)
    return -jnp.log(-jnp.log(u))


def _finish(b_x0, b_mask, b_eps, i_eps, x_flat, a_flat, mask_idx):
    eq = x_flat == mask_idx
    one_minus_a = jnp.float32(1.0) - a_flat
    p_x0 = a_flat + jnp.where(eq, one_minus_a, jnp.float32(0.0))
    p_m = jnp.where(eq, a_flat + one_minus_a, one_minus_a)
    v1 = _gumbel_from_bits(b_x0) + jnp.log(p_x0 + EPS)
    v2 = _gumbel_from_bits(b_mask) + jnp.log(p_m + EPS)
    v3 = _gumbel_from_bits(b_eps) + jnp.log(jnp.float32(0.0) + EPS)
    i1 = x_flat
    i2 = jnp.full_like(x_flat, mask_idx)
    best_v, best_i = v1, i1
    upd = (v2 > best_v) | ((v2 == best_v) & (i2 < best_i))
    best_v = jnp.where(upd, v2, best_v)
    best_i = jnp.where(upd, i2, best_i)
    upd = (v3 > best_v) | ((v3 == best_v) & (i_eps < best_i))
    best_i = jnp.where(upd, i_eps, best_i)
    return best_i


def kernel(structure, sequence, t):
    t_idx = jnp.arange(T + 1, dtype=jnp.float32)
    beta = 1.0 / (T - t_idx + 1.0)
    alpha = jnp.cumprod(1.0 - beta)
    key = jax.random.key(42)
    ks, kq = jax.random.split(key)
    kd_s = jax.random.key_data(ks)
    kd_q = jax.random.key_data(kq)
    B, L = structure.shape
    a_flat = jnp.repeat(alpha[t], L)
    outs = []
    for x, kd, n_cls, n_pad, mask_idx, lanes in (
            (structure, kd_s, STRUC_N, 520, STRUC_MASK, 128),
            (sequence, kd_q, SEQ_N, 40, SEQ_MASK, 512)):
        x_flat = x.reshape(-1).astype(jnp.int32)
        cands = _candidates(x_flat, kd, n_cls, n_pad, mask_idx, lanes)
        tok = _finish(*cands, x_flat, a_flat, mask_idx)
        outs.append(tok.reshape(B, L))
    return outs[0], outs[1], t


# speedup vs baseline: 1.5227x; 1.3584x over previous
import functools

import jax
import jax.numpy as jnp
import numpy as np
from jax.experimental import pallas as pl
from jax.experimental.pallas import tpu as pltpu

T = 500
STRUC_N = 517
SEQ_N = 33
STRUC_MASK = 516
SEQ_MASK = 32
EPS = 1e-10
_NEG = np.int32(-(2 ** 31))
_BIG = np.int32(2 ** 30)


def _threefry_biased(k1, k2, x1):
    ks0 = k1
    ks1 = k2
    ks2 = k1 ^ k2 ^ jnp.uint32(0x1BD11BDA)
    ks = (ks0, ks1, ks2)
    x0 = jnp.zeros_like(x1) + ks0
    x1 = x1 + ks1
    rot = (13, 15, 26, 6, 17, 29, 16, 24)
    rounds = (rot[0:4], rot[4:8], rot[0:4], rot[4:8], rot[0:4])
    for i, chunk in enumerate(rounds):
        for r in chunk:
            x0 = x0 + x1
            x1 = (x1 << jnp.uint32(r)) | (x1 >> jnp.uint32(32 - r))
            x1 = x0 ^ x1
        x0 = x0 + ks[(i + 1) % 3]
        x1 = x1 + ks[(i + 2) % 3] + jnp.uint32(i + 1)
    return jax.lax.bitcast_convert_type(x0 ^ x1 ^ jnp.uint32(0x80000000),
                                        jnp.int32)


def _combine(av, ai, bv, bi):
    take = (bv > av) | ((bv == av) & (bi < ai))
    return jnp.maximum(av, bv), jnp.where(take, bi, ai)


def _sample_body(key_ref, x0_ref, out_ref, *, n_cls, n_pad, chunk, mask_idx,
                 lanes):
    p = pl.program_id(0)
    k1 = jax.lax.bitcast_convert_type(key_ref[0], jnp.uint32)
    k2 = jax.lax.bitcast_convert_type(key_ref[1], jnp.uint32)
    x0 = x0_ref[0]
    lane1 = jax.lax.broadcasted_iota(jnp.int32, (1, lanes), 1)
    ibase1 = (p * lanes + lane1) * n_cls

    cand_i = jnp.concatenate(
        [ibase1 + x0, ibase1 + mask_idx,
         jnp.zeros((6, lanes), jnp.int32)], axis=0)
    cand_b = _threefry_biased(k1, k2, cand_i.astype(jnp.uint32))

    c_loc = jax.lax.broadcasted_iota(jnp.int32, (chunk, lanes), 0)
    lane = jax.lax.broadcasted_iota(jnp.int32, (chunk, lanes), 1)
    ibase = (p * lanes + lane) * n_cls + c_loc

    acc_v = jnp.full((chunk, lanes), _NEG, jnp.int32)
    acc_i = jnp.full((chunk, lanes), _BIG, jnp.int32)
    for c0 in range(0, n_pad, chunk):
        biased = _threefry_biased(k1, k2, (ibase + c0).astype(jnp.uint32))
        excl = c_loc == (x0 - c0)
        thr = n_cls - 1 - c0
        if thr < chunk:
            excl = excl | (c_loc >= thr)
        b_eps = jnp.where(excl, _NEG, biased)
        upd = b_eps > acc_v
        acc_v = jnp.maximum(acc_v, b_eps)
        acc_i = jnp.where(upd, c_loc + c0, acc_i)

    n = chunk
    while n > 1:
        h = n // 2
        mv, mi = _combine(acc_v[:h], acc_i[:h], acc_v[h:2 * h],
                          acc_i[h:2 * h])
        if n % 2:
            mv = jnp.concatenate([mv, acc_v[2 * h:n]], axis=0)
            mi = jnp.concatenate([mi, acc_i[2 * h:n]], axis=0)
        acc_v, acc_i = mv, mi
        n = h + (n % 2)

    out_ref[0, 0:1, :] = cand_b[0:1, :]
    out_ref[0, 1:2, :] = cand_b[1:2, :]
    out_ref[0, 2:3, :] = acc_v
    out_ref[0, 3:4, :] = acc_i
    out_ref[0, 4:8, :] = jnp.zeros((4, lanes), jnp.int32)


def _candidates(x_flat, key_data, n_cls, n_pad, chunk, mask_idx, lanes):
    rows = x_flat.shape[0]
    grid = rows // lanes
    x_in = x_flat.reshape(grid, 1, lanes)
    body = functools.partial(_sample_body, n_cls=n_cls, n_pad=n_pad,
                             chunk=chunk, mask_idx=mask_idx, lanes=lanes)
    out = pl.pallas_call(
        body,
        grid=(grid,),
        in_specs=[
            pl.BlockSpec(memory_space=pltpu.SMEM),
            pl.BlockSpec((1, 1, lanes), lambda p: (p, 0, 0)),
        ],
        out_specs=pl.BlockSpec((1, 8, lanes), lambda p: (p, 0, 0)),
        out_shape=jax.ShapeDtypeStruct((grid, 8, lanes), jnp.int32),
        compiler_params=pltpu.CompilerParams(
            dimension_semantics=("parallel",)),
    )(key_data.astype(jnp.int32), x_in)
    unbias = lambda b: jax.lax.bitcast_convert_type(b, jnp.uint32) ^ jnp.uint32(
        0x80000000)
    b_x0 = unbias(out[:, 0, :].reshape(rows))
    b_mask = unbias(out[:, 1, :].reshape(rows))
    b_eps = unbias(out[:, 2, :].reshape(rows))
    i_eps = out[:, 3, :].reshape(rows)
    return b_x0, b_mask, b_eps, i_eps


def _gumbel_from_bits(bits):
    tiny = jnp.float32(jnp.finfo(jnp.float32).tiny)
    fb = (bits >> jnp.uint32(9)) | jnp.uint32(0x3F800000)
    floats = jax.lax.bitcast_convert_type(fb, jnp.float32) - jnp.float32(1.0)
    u = jnp.maximum(tiny, floats * (jnp.float32(1.0) - tiny) + tiny)
    return -jnp.log(-jnp.log(u))


def _finish(b_x0, b_mask, b_eps, i_eps, x_flat, a_flat, mask_idx):
    eq = x_flat == mask_idx
    one_minus_a = jnp.float32(1.0) - a_flat
    p_x0 = a_flat + jnp.where(eq, one_minus_a, jnp.float32(0.0))
    p_m = jnp.where(eq, a_flat + one_minus_a, one_minus_a)
    v1 = _gumbel_from_bits(b_x0) + jnp.log(p_x0 + EPS)
    v2 = _gumbel_from_bits(b_mask) + jnp.log(p_m + EPS)
    v3 = _gumbel_from_bits(b_eps) + jnp.log(jnp.float32(0.0) + EPS)
    i1 = x_flat
    i2 = jnp.full_like(x_flat, mask_idx)
    best_v, best_i = v1, i1
    upd = (v2 > best_v) | ((v2 == best_v) & (i2 < best_i))
    best_v = jnp.where(upd, v2, best_v)
    best_i = jnp.where(upd, i2, best_i)
    upd = (v3 > best_v) | ((v3 == best_v) & (i_eps < best_i))
    best_i = jnp.where(upd, i_eps, best_i)
    return best_i


def kernel(structure, sequence, t):
    t_idx = jnp.arange(T + 1, dtype=jnp.float32)
    beta = 1.0 / (T - t_idx + 1.0)
    alpha = jnp.cumprod(1.0 - beta)
    key = jax.random.key(42)
    ks, kq = jax.random.split(key)
    kd_s = jax.random.key_data(ks)
    kd_q = jax.random.key_data(kq)
    B, L = structure.shape
    a_flat = jnp.repeat(alpha[t], L)
    outs = []
    for x, kd, n_cls, n_pad, chunk, mask_idx, lanes in (
            (structure, kd_s, STRUC_N, 520, 40, STRUC_MASK, 128),
            (sequence, kd_q, SEQ_N, 40, 40, SEQ_MASK, 512)):
        x_flat = x.reshape(-1).astype(jnp.int32)
        cands = _candidates(x_flat, kd, n_cls, n_pad, chunk, mask_idx, lanes)
        tok = _finish(*cands, x_flat, a_flat, mask_idx)
        outs.append(tok.reshape(B, L))
    return outs[0], outs[1], t


# speedup vs baseline: 1.5451x; 1.0147x over previous
import jax
import jax.numpy as jnp
import numpy as np
from jax.experimental import pallas as pl
from jax.experimental.pallas import tpu as pltpu

T = 500
STRUC_N = 517
SEQ_N = 33
STRUC_MASK = 516
SEQ_MASK = 32
EPS = 1e-10
_NEG = np.int32(-(2 ** 31))
_BIG = np.int32(2 ** 30)
_LANES = 128
_CHUNK = 40


def _threefry_biased(k1, k2, x1):
    ks2 = k1 ^ k2 ^ jnp.uint32(0x1BD11BDA)
    ks = (k1, k2, ks2)
    x0 = jnp.zeros_like(x1) + k1
    x1 = x1 + k2
    rot = (13, 15, 26, 6, 17, 29, 16, 24)
    rounds = (rot[0:4], rot[4:8], rot[0:4], rot[4:8], rot[0:4])
    for i, chunk in enumerate(rounds):
        for r in chunk:
            x0 = x0 + x1
            x1 = (x1 << jnp.uint32(r)) | (x1 >> jnp.uint32(32 - r))
            x1 = x0 ^ x1
        x0 = x0 + ks[(i + 1) % 3]
        x1 = x1 + ks[(i + 2) % 3] + jnp.uint32(i + 1)
    return jax.lax.bitcast_convert_type(x0 ^ x1 ^ jnp.uint32(0x80000000),
                                        jnp.int32)


def _combine(av, ai, bv, bi):
    take = (bv > av) | ((bv == av) & (bi < ai))
    return jnp.maximum(av, bv), jnp.where(take, bi, ai)


def _tail_scan(k1, k2, ibase, c_loc, x0, n_cls, n_pad):
    acc_v = jnp.full((_CHUNK, _LANES), _NEG, jnp.int32)
    acc_i = jnp.full((_CHUNK, _LANES), _BIG, jnp.int32)
    for c0 in range(0, n_pad, _CHUNK):
        biased = _threefry_biased(k1, k2, (ibase + c0).astype(jnp.uint32))
        excl = c_loc == (x0 - c0)
        thr = n_cls - 1 - c0
        if thr < _CHUNK:
            excl = excl | (c_loc >= thr)
        b_eps = jnp.where(excl, _NEG, biased)
        upd = b_eps > acc_v
        acc_v = jnp.maximum(acc_v, b_eps)
        acc_i = jnp.where(upd, c_loc + c0, acc_i)
    n = _CHUNK
    while n > 1:
        h = n // 2
        mv, mi = _combine(acc_v[:h], acc_i[:h], acc_v[h:2 * h],
                          acc_i[h:2 * h])
        if n % 2:
            mv = jnp.concatenate([mv, acc_v[2 * h:n]], axis=0)
            mi = jnp.concatenate([mi, acc_i[2 * h:n]], axis=0)
        acc_v, acc_i = mv, mi
        n = h + (n % 2)
    return acc_v, acc_i


def _both_body(keys_ref, xs_ref, xq_ref, out_ref):
    p = pl.program_id(0)
    u32 = lambda v: jax.lax.bitcast_convert_type(v, jnp.uint32)
    k1s, k2s = u32(keys_ref[0]), u32(keys_ref[1])
    k1q, k2q = u32(keys_ref[2]), u32(keys_ref[3])
    xs = xs_ref[0]
    xq = xq_ref[0]
    lane1 = jax.lax.broadcasted_iota(jnp.int32, (1, _LANES), 1)
    row1 = p * _LANES + lane1
    ibs1 = row1 * STRUC_N
    ibq1 = row1 * SEQ_N

    cand_i = jnp.concatenate(
        [ibs1 + xs, ibs1 + STRUC_MASK, ibq1 + xq, ibq1 + SEQ_MASK,
         jnp.zeros((4, _LANES), jnp.int32)], axis=0)
    srow = jax.lax.broadcasted_iota(jnp.int32, (8, 1), 0) < 2
    ck1 = jnp.where(srow, k1s, k1q)
    ck2 = jnp.where(srow, k2s, k2q)
    cand_b = _threefry_biased(ck1, ck2, cand_i.astype(jnp.uint32))

    c_loc = jax.lax.broadcasted_iota(jnp.int32, (_CHUNK, _LANES), 0)
    lane = jax.lax.broadcasted_iota(jnp.int32, (_CHUNK, _LANES), 1)
    row = p * _LANES + lane
    sv, si = _tail_scan(k1s, k2s, row * STRUC_N + c_loc, c_loc, xs,
                        STRUC_N, 520)
    qv, qi = _tail_scan(k1q, k2q, row * SEQ_N + c_loc, c_loc, xq,
                        SEQ_N, 40)

    out_ref[0, 0:1, :] = cand_b[0:1, :]
    out_ref[0, 1:2, :] = cand_b[1:2, :]
    out_ref[0, 2:3, :] = sv
    out_ref[0, 3:4, :] = si
    out_ref[0, 4:5, :] = cand_b[2:3, :]
    out_ref[0, 5:6, :] = cand_b[3:4, :]
    out_ref[0, 6:7, :] = qv
    out_ref[0, 7:8, :] = qi


def _gumbel_from_bits(bits):
    tiny = jnp.float32(jnp.finfo(jnp.float32).tiny)
    fb = (bits >> jnp.uint32(9)) | jnp.uint32(0x3F800000)
    floats = jax.lax.bitcast_convert_type(fb, jnp.float32) - jnp.float32(1.0)
    u = jnp.maximum(tiny, floats * (jnp.float32(1.0) - tiny) + tiny)
    return -jnp.log(-jnp.log(u))


def _finish(b_x0, b_mask, b_eps, i_eps, x_flat, a_flat, mask_idx):
    eq = x_flat == mask_idx
    one_minus_a = jnp.float32(1.0) - a_flat
    p_x0 = a_flat + jnp.where(eq, one_minus_a, jnp.float32(0.0))
    p_m = jnp.where(eq, a_flat + one_minus_a, one_minus_a)
    v1 = _gumbel_from_bits(b_x0) + jnp.log(p_x0 + EPS)
    v2 = _gumbel_from_bits(b_mask) + jnp.log(p_m + EPS)
    v3 = _gumbel_from_bits(b_eps) + jnp.log(jnp.float32(0.0) + EPS)
    i1 = x_flat
    i2 = jnp.full_like(x_flat, mask_idx)
    best_v, best_i = v1, i1
    upd = (v2 > best_v) | ((v2 == best_v) & (i2 < best_i))
    best_v = jnp.where(upd, v2, best_v)
    best_i = jnp.where(upd, i2, best_i)
    upd = (v3 > best_v) | ((v3 == best_v) & (i_eps < best_i))
    best_i = jnp.where(upd, i_eps, best_i)
    return best_i


def kernel(structure, sequence, t):
    t_idx = jnp.arange(T + 1, dtype=jnp.float32)
    beta = 1.0 / (T - t_idx + 1.0)
    alpha = jnp.cumprod(1.0 - beta)
    key = jax.random.key(42)
    ks, kq = jax.random.split(key)
    keys = jnp.concatenate([jax.random.key_data(ks),
                            jax.random.key_data(kq)]).astype(jnp.int32)
    B, L = structure.shape
    rows = B * L
    grid = rows // _LANES
    xs = structure.reshape(grid, 1, _LANES).astype(jnp.int32)
    xq = sequence.reshape(grid, 1, _LANES).astype(jnp.int32)
    out = pl.pallas_call(
        _both_body,
        grid=(grid,),
        in_specs=[
            pl.BlockSpec(memory_space=pltpu.SMEM),
            pl.BlockSpec((1, 1, _LANES), lambda p: (p, 0, 0)),
            pl.BlockSpec((1, 1, _LANES), lambda p: (p, 0, 0)),
        ],
        out_specs=pl.BlockSpec((1, 8, _LANES), lambda p: (p, 0, 0)),
        out_shape=jax.ShapeDtypeStruct((grid, 8, _LANES), jnp.int32),
        compiler_params=pltpu.CompilerParams(
            dimension_semantics=("parallel",)),
    )(keys, xs, xq)
    unbias = lambda b: jax.lax.bitcast_convert_type(b, jnp.uint32) ^ jnp.uint32(
        0x80000000)
    a_flat = jnp.repeat(alpha[t], L)
    outs = []
    for x, base, mask_idx in ((structure, 0, STRUC_MASK),
                              (sequence, 4, SEQ_MASK)):
        x_flat = x.reshape(-1).astype(jnp.int32)
        b_x0 = unbias(out[:, base + 0, :].reshape(rows))
        b_mask = unbias(out[:, base + 1, :].reshape(rows))
        b_eps = unbias(out[:, base + 2, :].reshape(rows))
        i_eps = out[:, base + 3, :].reshape(rows)
        tok = _finish(b_x0, b_mask, b_eps, i_eps, x_flat, a_flat, mask_idx)
        outs.append(tok.reshape(B, L))
    return outs[0], outs[1], t


# speedup vs baseline: 1.5819x; 1.0238x over previous
import jax
import jax.numpy as jnp
import numpy as np
from jax.experimental import pallas as pl
from jax.experimental.pallas import tpu as pltpu

T = 500
STRUC_N = 517
SEQ_N = 33
STRUC_MASK = 516
SEQ_MASK = 32
EPS = 1e-10
_NEG = np.int32(-(2 ** 31))
_BIG = np.int32(2 ** 30)
_LANES = 128
_CHUNK = 40


def _threefry_biased(k1, k2, x1):
    ks2 = k1 ^ k2 ^ jnp.uint32(0x1BD11BDA)
    ks = (k1, k2, ks2)
    x0 = jnp.zeros_like(x1) + k1
    x1 = x1 + k2
    rot = (13, 15, 26, 6, 17, 29, 16, 24)
    rounds = (rot[0:4], rot[4:8], rot[0:4], rot[4:8], rot[0:4])
    for i, chunk in enumerate(rounds):
        for r in chunk:
            x0 = x0 + x1
            x1 = (x1 << jnp.uint32(r)) | (x1 >> jnp.uint32(32 - r))
            x1 = x0 ^ x1
        x0 = x0 + ks[(i + 1) % 3]
        x1 = x1 + ks[(i + 2) % 3] + jnp.uint32(i + 1)
    return jax.lax.bitcast_convert_type(x0 ^ x1 ^ jnp.uint32(0x80000000),
                                        jnp.int32)


def _combine(av, ai, bv, bi):
    take = (bv > av) | ((bv == av) & (bi < ai))
    return jnp.maximum(av, bv), jnp.where(take, bi, ai)


def _tail_scan(k1, k2, ibase, c_loc, x0, n_cls, n_pad):
    acc_v = jnp.full((_CHUNK, _LANES), _NEG, jnp.int32)
    acc_i = jnp.full((_CHUNK, _LANES), _BIG, jnp.int32)
    for c0 in range(0, n_pad, _CHUNK):
        biased = _threefry_biased(k1, k2, (ibase + c0).astype(jnp.uint32))
        excl = c_loc == (x0 - c0)
        thr = n_cls - 1 - c0
        if thr < _CHUNK:
            excl = excl | (c_loc >= thr)
        b_eps = jnp.where(excl, _NEG, biased)
        upd = b_eps > acc_v
        acc_v = jnp.maximum(acc_v, b_eps)
        acc_i = jnp.where(upd, c_loc + c0, acc_i)
    n = _CHUNK
    while n > 1:
        h = n // 2
        mv, mi = _combine(acc_v[:h], acc_i[:h], acc_v[h:2 * h],
                          acc_i[h:2 * h])
        if n % 2:
            mv = jnp.concatenate([mv, acc_v[2 * h:n]], axis=0)
            mi = jnp.concatenate([mi, acc_i[2 * h:n]], axis=0)
        acc_v, acc_i = mv, mi
        n = h + (n % 2)
    return acc_v, acc_i


def _both_body(keys_ref, xs_ref, xq_ref, a_ref, out_ref):
    p = pl.program_id(0)
    u32 = lambda v: jax.lax.bitcast_convert_type(v, jnp.uint32)
    k1s, k2s = u32(keys_ref[0]), u32(keys_ref[1])
    k1q, k2q = u32(keys_ref[2]), u32(keys_ref[3])
    xs = xs_ref[0]
    xq = xq_ref[0]
    lane1 = jax.lax.broadcasted_iota(jnp.int32, (1, _LANES), 1)
    row1 = p * _LANES + lane1
    ibs1 = row1 * STRUC_N
    ibq1 = row1 * SEQ_N

    cand_i = jnp.concatenate(
        [ibs1 + xs, ibs1 + STRUC_MASK, ibq1 + xq, ibq1 + SEQ_MASK,
         jnp.zeros((4, _LANES), jnp.int32)], axis=0)
    srow = jax.lax.broadcasted_iota(jnp.int32, (8, 1), 0) < 2
    ck1 = jnp.where(srow, k1s, k1q)
    ck2 = jnp.where(srow, k2s, k2q)
    cand_b = _threefry_biased(ck1, ck2, cand_i.astype(jnp.uint32))

    c_loc = jax.lax.broadcasted_iota(jnp.int32, (_CHUNK, _LANES), 0)
    lane = jax.lax.broadcasted_iota(jnp.int32, (_CHUNK, _LANES), 1)
    row = p * _LANES + lane
    sv, si = _tail_scan(k1s, k2s, row * STRUC_N + c_loc, c_loc, xs,
                        STRUC_N, 520)
    qv, qi = _tail_scan(k1q, k2q, row * SEQ_N + c_loc, c_loc, xq,
                        SEQ_N, 40)

    a = a_ref[0]
    unb = lambda b: jax.lax.bitcast_convert_type(b, jnp.uint32) ^ jnp.uint32(
        0x80000000)
    tok_s = _finish(unb(cand_b[0:1, :]), unb(cand_b[1:2, :]), unb(sv), si,
                    xs, a, STRUC_MASK)
    tok_q = _finish(unb(cand_b[2:3, :]), unb(cand_b[3:4, :]), unb(qv), qi,
                    xq, a, SEQ_MASK)
    out_ref[0, 0:1, :] = tok_s
    out_ref[0, 1:2, :] = tok_q
    out_ref[0, 2:8, :] = jnp.zeros((6, _LANES), jnp.int32)


def _gumbel_from_bits(bits):
    tiny = jnp.float32(jnp.finfo(jnp.float32).tiny)
    fb = (bits >> jnp.uint32(9)) | jnp.uint32(0x3F800000)
    floats = jax.lax.bitcast_convert_type(fb, jnp.float32) - jnp.float32(1.0)
    u = jnp.maximum(tiny, floats * (jnp.float32(1.0) - tiny) + tiny)
    return -jnp.log(-jnp.log(u))


def _finish(b_x0, b_mask, b_eps, i_eps, x_flat, a_flat, mask_idx):
    eq = x_flat == mask_idx
    one_minus_a = jnp.float32(1.0) - a_flat
    p_x0 = a_flat + jnp.where(eq, one_minus_a, jnp.float32(0.0))
    p_m = jnp.where(eq, a_flat + one_minus_a, one_minus_a)
    v1 = _gumbel_from_bits(b_x0) + jnp.log(p_x0 + EPS)
    v2 = _gumbel_from_bits(b_mask) + jnp.log(p_m + EPS)
    v3 = _gumbel_from_bits(b_eps) + jnp.log(jnp.float32(0.0) + EPS)
    i1 = x_flat
    i2 = jnp.full_like(x_flat, mask_idx)
    best_v, best_i = v1, i1
    upd = (v2 > best_v) | ((v2 == best_v) & (i2 < best_i))
    best_v = jnp.where(upd, v2, best_v)
    best_i = jnp.where(upd, i2, best_i)
    upd = (v3 > best_v) | ((v3 == best_v) & (i_eps < best_i))
    best_i = jnp.where(upd, i_eps, best_i)
    return best_i


def kernel(structure, sequence, t):
    t_idx = jnp.arange(T + 1, dtype=jnp.float32)
    beta = 1.0 / (T - t_idx + 1.0)
    alpha = jnp.cumprod(1.0 - beta)
    key = jax.random.key(42)
    ks, kq = jax.random.split(key)
    keys = jnp.concatenate([jax.random.key_data(ks),
                            jax.random.key_data(kq)]).astype(jnp.int32)
    B, L = structure.shape
    rows = B * L
    grid = rows // _LANES
    xs = structure.reshape(grid, 1, _LANES).astype(jnp.int32)
    xq = sequence.reshape(grid, 1, _LANES).astype(jnp.int32)
    a_in = jnp.repeat(alpha[t], L).reshape(grid, 1, _LANES)
    out = pl.pallas_call(
        _both_body,
        grid=(grid,),
        in_specs=[
            pl.BlockSpec(memory_space=pltpu.SMEM),
            pl.BlockSpec((1, 1, _LANES), lambda p: (p, 0, 0)),
            pl.BlockSpec((1, 1, _LANES), lambda p: (p, 0, 0)),
            pl.BlockSpec((1, 1, _LANES), lambda p: (p, 0, 0)),
        ],
        out_specs=pl.BlockSpec((1, 8, _LANES), lambda p: (p, 0, 0)),
        out_shape=jax.ShapeDtypeStruct((grid, 8, _LANES), jnp.int32),
        compiler_params=pltpu.CompilerParams(
            dimension_semantics=("parallel",)),
    )(keys, xs, xq, a_in)
    return (out[:, 0, :].reshape(B, L), out[:, 1, :].reshape(B, L), t)
